# baseline (speedup 1.0000x reference)
import functools

import jax
import jax.numpy as jnp
from jax import lax
from jax.experimental import pallas as pl
from jax.experimental.pallas import tpu as pltpu
from jax.experimental.pallas import tpu_sc as plsc

N = 10000
NN = 16
D = 128
K = 15
RADIUS = 1.0
EPS = 1e-5
NEG_SLOPE = 0.2

NE = N * NN
B = 400
NBLK = N // B
B3 = 1000
CHUNK = 200


def _leaky(v):
    return jnp.where(v >= 0, v, NEG_SLOPE * v)


@functools.cache
def _sc_gathers():
    info = plsc.get_sparse_core_info()
    nc = info.num_cores
    nw = nc * info.num_subcores
    bpw = NE // nw
    nchunk = bpw // CHUNK
    mesh = plsc.VectorSubcoreMesh(core_axis_name="c", subcore_axis_name="s")

    def pipeline(idx_hbm, idx_v, sem_i, tables, wid):
        pltpu.async_copy(idx_hbm.at[pl.ds(wid * bpw, bpw)], idx_v,
                         sem_i).wait()

        def gather(t):
            r = t % 3
            hs = []
            for (src, _out, bufs, gs, _wsems) in tables:
                hs.append(pltpu.async_copy(
                    src.at[idx_v.at[pl.ds(t * CHUNK, CHUNK)]], bufs[r],
                    gs[r]))
            return hs

        g = {0: gather(0), 1: gather(1)}
        w = {}
        for t in range(nchunk):
            r = t % 3
            if t + 2 < nchunk:
                if t >= 1:
                    for h in w[t - 1]:
                        h.wait()
                g[t + 2] = gather(t + 2)
            for h in g[t]:
                h.wait()
            base = wid * bpw + t * CHUNK
            w[t] = [
                pltpu.async_copy(bufs[r], out.at[pl.ds(base, CHUNK)],
                                 wsems[r])
                for (_src, out, bufs, _gs, wsems) in tables
            ]
        for t in (nchunk - 2, nchunk - 1):
            for h in w[t]:
                h.wait()

    @functools.partial(
        pl.kernel,
        mesh=mesh,
        compiler_params=pltpu.CompilerParams(use_tc_tiling_on_sc=False),
        out_type=(
            jax.ShapeDtypeStruct((NE, D), jnp.float32),
            jax.ShapeDtypeStruct((NE, 16), jnp.float32),
        ),
        scratch_types=(
            [pltpu.VMEM((bpw,), jnp.int32)]
            + [pltpu.VMEM((CHUNK, D), jnp.float32)] * 3
            + [pltpu.VMEM((CHUNK, 16), jnp.float32)] * 3
            + [pltpu.SemaphoreType.DMA] * 13
        ),
    )
    def gather_l1(feat_hbm, pos_hbm, idx_hbm, feat_out, pos_out,
                  idx_v, f0, f1, f2, p0, p1, p2,
                  sem_i, gf0, gf1, gf2, gp0, gp1, gp2,
                  wf0, wf1, wf2, wp0, wp1, wp2):
        wid = lax.axis_index("s") * nc + lax.axis_index("c")
        pipeline(idx_hbm, idx_v, sem_i,
                 [(feat_hbm, feat_out, [f0, f1, f2],
                   [gf0, gf1, gf2], [wf0, wf1, wf2]),
                  (pos_hbm, pos_out, [p0, p1, p2],
                   [gp0, gp1, gp2], [wp0, wp1, wp2])],
                 wid)

    @functools.partial(
        pl.kernel,
        mesh=mesh,
        out_type=jax.ShapeDtypeStruct((NE, D), jnp.float32),
        scratch_types=(
            [pltpu.VMEM((bpw,), jnp.int32)]
            + [pltpu.VMEM((CHUNK, D), jnp.float32)] * 3
            + [pltpu.SemaphoreType.DMA] * 7
        ),
    )
    def gather_l2(feat_hbm, idx_hbm, feat_out,
                  idx_v, f0, f1, f2,
                  sem_i, gf0, gf1, gf2, wf0, wf1, wf2):
        wid = lax.axis_index("s") * nc + lax.axis_index("c")
        pipeline(idx_hbm, idx_v, sem_i,
                 [(feat_hbm, feat_out, [f0, f1, f2],
                   [gf0, gf1, gf2], [wf0, wf1, wf2])],
                 wid)

    return gather_l1, gather_l2


def _write_stats(stats_ref, acc):
    stats_ref[0, 0:1, :] = jnp.sum(acc, axis=0, keepdims=True)
    stats_ref[0, 1:2, :] = jnp.sum(acc * acc, axis=0, keepdims=True)
    stats_ref[0, 2:8, :] = jnp.zeros((6, D), jnp.float32)


def _scale_shift(stats, gamma, beta):
    tot = jnp.sum(stats, axis=0)
    mean = tot[0:1, :] * (1.0 / N)
    ex2 = tot[1:2, :] * (1.0 / N)
    var = ex2 - mean * mean
    scale = gamma / jnp.sqrt(var + EPS)
    shift = beta - mean * scale
    return scale, shift


BE = B * NN


NG = B // 8


def _conv_weighted(gpos_ref, qrep_ref, m_ref, kp2_ref, sel_ref, w_ref,
                   feats, out_ref, stats_ref, ws_ref):
    qrep = jnp.broadcast_to(qrep_ref[...][:, None, :],
                            (B, NN, 16)).reshape(BE, 16)
    rel = gpos_ref[...] - qrep
    lhs = jnp.concatenate([rel * rel, rel], axis=1)
    d2 = jnp.dot(lhs, m_ref[...],
                 preferred_element_type=jnp.float32) + kp2_ref[...]
    d2c = jnp.maximum(d2, 1e-24)
    r = lax.rsqrt(d2c)
    r = r * (1.5 - 0.5 * d2c * r * r)
    dist = d2c * r
    infl = jnp.maximum(0.0, 1.0 - dist)
    mask = sel_ref[...]
    for g in range(NG):
        st = infl[g * 128:(g + 1) * 128, :] * mask
        fg = feats[g * 128:(g + 1) * 128, :]
        wg = jax.lax.dot_general(
            st, fg, (((0,), (0,)), ((), ())),
            preferred_element_type=jnp.float32)
        for k in range(K):
            ws_ref[k, g * 8:(g + 1) * 8, :] = wg[k * 8:(k + 1) * 8, :]
    acc = jnp.dot(ws_ref[0], w_ref[0], preferred_element_type=jnp.float32)
    for k in range(1, K):
        acc = acc + jnp.dot(ws_ref[k], w_ref[k],
                            preferred_element_type=jnp.float32)
    out_ref[...] = acc
    _write_stats(stats_ref, acc)


def _conv1_body(gpos_ref, qrep_ref, gfeat_ref, m_ref, kp2_ref, sel_ref, w_ref,
                out_ref, stats_ref, ws_ref):
    _conv_weighted(gpos_ref, qrep_ref, m_ref, kp2_ref, sel_ref, w_ref,
                   gfeat_ref[...], out_ref, stats_ref, ws_ref)


def _conv2_body(gpos_ref, qrep_ref, gfeat_ref, m_ref, kp2_ref, sel_ref, w_ref,
                stats_in_ref, gamma_ref, beta_ref,
                out_ref, stats_ref, ws_ref):
    scale, shift = _scale_shift(stats_in_ref[...], gamma_ref[...], beta_ref[...])
    feats = _leaky(gfeat_ref[...] * scale + shift)
    _conv_weighted(gpos_ref, qrep_ref, m_ref, kp2_ref, sel_ref, w_ref,
                   feats, out_ref, stats_ref, ws_ref)


_CONV_COMMON_SPECS = [
    pl.BlockSpec((BE, 16), lambda i: (i, 0)),
    pl.BlockSpec((B, 16), lambda i: (i, 0)),
    pl.BlockSpec((BE, D), lambda i: (i, 0)),
    pl.BlockSpec((32, D), lambda i: (0, 0)),
    pl.BlockSpec((1, D), lambda i: (0, 0)),
    pl.BlockSpec((D, D), lambda i: (0, 0)),
    pl.BlockSpec((K, D, D), lambda i: (0, 0, 0)),
]

_CONV_OUT_SPECS = [
    pl.BlockSpec((B, D), lambda i: (i, 0)),
    pl.BlockSpec((1, 8, D), lambda i: (i, 0, 0)),
]

_CONV_OUT_SHAPE = [
    jax.ShapeDtypeStruct((N, D), jnp.float32),
    jax.ShapeDtypeStruct((NBLK, 8, D), jnp.float32),
]


def _tc_conv1(gpos, qrep, gfeat, m, kp2, sel, wflat):
    return pl.pallas_call(
        _conv1_body,
        grid=(NBLK,),
        in_specs=_CONV_COMMON_SPECS,
        out_specs=_CONV_OUT_SPECS,
        out_shape=_CONV_OUT_SHAPE,
        scratch_shapes=[pltpu.VMEM((K, B, D), jnp.float32)],
    )(gpos, qrep, gfeat, m, kp2, sel, wflat)


def _tc_conv2(gpos, qrep, gfeat, m, kp2, sel, wflat, stats_in, gamma, beta):
    return pl.pallas_call(
        _conv2_body,
        grid=(NBLK,),
        in_specs=_CONV_COMMON_SPECS + [
            pl.BlockSpec((NBLK, 8, D), lambda i: (0, 0, 0)),
            pl.BlockSpec((1, D), lambda i: (0, 0)),
            pl.BlockSpec((1, D), lambda i: (0, 0)),
        ],
        out_specs=_CONV_OUT_SPECS,
        out_shape=_CONV_OUT_SHAPE,
        scratch_shapes=[pltpu.VMEM((K, B, D), jnp.float32)],
    )(gpos, qrep, gfeat, m, kp2, sel, wflat, stats_in, gamma, beta)


def _final_body(h_ref, x_ref, stats_in_ref, gamma_ref, beta_ref, out_ref):
    scale, shift = _scale_shift(stats_in_ref[...], gamma_ref[...], beta_ref[...])
    out_ref[...] = _leaky(h_ref[...] * scale + shift) + x_ref[...]


def _tc_final(h, x, stats_in, gamma, beta):
    return pl.pallas_call(
        _final_body,
        grid=(N // B3,),
        in_specs=[
            pl.BlockSpec((B3, D), lambda i: (i, 0)),
            pl.BlockSpec((B3, D), lambda i: (i, 0)),
            pl.BlockSpec((NBLK, 8, D), lambda i: (0, 0, 0)),
            pl.BlockSpec((1, D), lambda i: (0, 0)),
            pl.BlockSpec((1, D), lambda i: (0, 0)),
        ],
        out_specs=pl.BlockSpec((B3, D), lambda i: (i, 0)),
        out_shape=jax.ShapeDtypeStruct((N, D), jnp.float32),
    )(h, x, stats_in, gamma, beta)


def kernel(x, points, neighbors, kernel_points, W0, W1,
           gamma0, beta0, gamma1, beta1):
    idx = neighbors.astype(jnp.int32).reshape(-1)
    qpts = jnp.pad(points, ((0, 0), (0, 13)))
    kp = jnp.pad(kernel_points, ((0, 1), (0, 13)))
    m = jnp.concatenate([jnp.ones((16, 16), jnp.float32), -2.0 * kp.T], axis=0)
    m = jnp.repeat(m, 8, axis=1)
    kp2 = jnp.repeat(jnp.sum(kp * kp, axis=1).reshape(1, 16), 8, axis=1)
    lane = jnp.arange(D)
    edge = jnp.arange(D)
    sel = ((lane[None, :] % 8 == edge[:, None] // NN) &
           (lane[None, :] // 8 < K)).astype(jnp.float32)
    w0f = W0
    w1f = W1
    g0 = gamma0.reshape(1, D)
    b0 = beta0.reshape(1, D)
    g1 = gamma1.reshape(1, D)
    b1 = beta1.reshape(1, D)

    gather_l1, gather_l2 = _sc_gathers()
    gfeat0, gpos = gather_l1(x, qpts, idx)

    h1_raw, stats1 = _tc_conv1(gpos, qpts, gfeat0, m, kp2, sel, w0f)

    gfeat1 = gather_l2(h1_raw, idx)

    h2_raw, stats2 = _tc_conv2(gpos, qpts, gfeat1, m, kp2, sel, w1f,
                               stats1, g0, b0)

    return _tc_final(h2_raw, x, stats2, g1, b1)


# speedup vs baseline: 1.0301x; 1.0301x over previous
import functools

import jax
import jax.numpy as jnp
from jax import lax
from jax.experimental import pallas as pl
from jax.experimental.pallas import tpu as pltpu
from jax.experimental.pallas import tpu_sc as plsc

N = 10000
NN = 16
D = 128
K = 15
RADIUS = 1.0
EPS = 1e-5
NEG_SLOPE = 0.2

NE = N * NN
B = 400
NBLK = N // B
B3 = 1000
CHUNK = 200


def _leaky(v):
    return jnp.where(v >= 0, v, NEG_SLOPE * v)


@functools.cache
def _sc_gathers():
    info = plsc.get_sparse_core_info()
    nc = info.num_cores
    nw = nc * info.num_subcores
    bpw = NE // nw
    nchunk = bpw // CHUNK
    mesh = plsc.VectorSubcoreMesh(core_axis_name="c", subcore_axis_name="s")

    def pipeline(idx_hbm, idx_v, sem_i, tables, wid):
        pltpu.async_copy(idx_hbm.at[pl.ds(wid * bpw, bpw)], idx_v,
                         sem_i).wait()

        def gather(t):
            r = t % 3
            hs = []
            for (src, _out, bufs, gs, _wsems) in tables:
                hs.append(pltpu.async_copy(
                    src.at[idx_v.at[pl.ds(t * CHUNK, CHUNK)]], bufs[r],
                    gs[r]))
            return hs

        g = {0: gather(0), 1: gather(1)}
        w = {}
        for t in range(nchunk):
            r = t % 3
            if t + 2 < nchunk:
                if t >= 1:
                    for h in w[t - 1]:
                        h.wait()
                g[t + 2] = gather(t + 2)
            for h in g[t]:
                h.wait()
            base = wid * bpw + t * CHUNK
            w[t] = [
                pltpu.async_copy(bufs[r], out.at[pl.ds(base, CHUNK)],
                                 wsems[r])
                for (_src, out, bufs, _gs, wsems) in tables
            ]
        for t in (nchunk - 2, nchunk - 1):
            for h in w[t]:
                h.wait()

    @functools.partial(
        pl.kernel,
        mesh=mesh,
        compiler_params=pltpu.CompilerParams(use_tc_tiling_on_sc=False),
        out_type=(
            jax.ShapeDtypeStruct((NE, D), jnp.float32),
            jax.ShapeDtypeStruct((NE, 16), jnp.float32),
        ),
        scratch_types=(
            [pltpu.VMEM((bpw,), jnp.int32)]
            + [pltpu.VMEM((CHUNK, D), jnp.float32)] * 3
            + [pltpu.VMEM((CHUNK, 16), jnp.float32)] * 3
            + [pltpu.SemaphoreType.DMA] * 13
        ),
    )
    def gather_l1(feat_hbm, pos_hbm, idx_hbm, feat_out, pos_out,
                  idx_v, f0, f1, f2, p0, p1, p2,
                  sem_i, gf0, gf1, gf2, gp0, gp1, gp2,
                  wf0, wf1, wf2, wp0, wp1, wp2):
        wid = lax.axis_index("s") * nc + lax.axis_index("c")
        pipeline(idx_hbm, idx_v, sem_i,
                 [(feat_hbm, feat_out, [f0, f1, f2],
                   [gf0, gf1, gf2], [wf0, wf1, wf2]),
                  (pos_hbm, pos_out, [p0, p1, p2],
                   [gp0, gp1, gp2], [wp0, wp1, wp2])],
                 wid)

    @functools.partial(
        pl.kernel,
        mesh=mesh,
        out_type=jax.ShapeDtypeStruct((NE, D), jnp.float32),
        scratch_types=(
            [pltpu.VMEM((bpw,), jnp.int32)]
            + [pltpu.VMEM((CHUNK, D), jnp.float32)] * 3
            + [pltpu.SemaphoreType.DMA] * 7
        ),
    )
    def gather_l2(feat_hbm, idx_hbm, feat_out,
                  idx_v, f0, f1, f2,
                  sem_i, gf0, gf1, gf2, wf0, wf1, wf2):
        wid = lax.axis_index("s") * nc + lax.axis_index("c")
        pipeline(idx_hbm, idx_v, sem_i,
                 [(feat_hbm, feat_out, [f0, f1, f2],
                   [gf0, gf1, gf2], [wf0, wf1, wf2])],
                 wid)

    return gather_l1, gather_l2


def _write_stats(stats_ref, acc):
    stats_ref[0, 0:1, :] = jnp.sum(acc, axis=0, keepdims=True)
    stats_ref[0, 1:2, :] = jnp.sum(acc * acc, axis=0, keepdims=True)
    stats_ref[0, 2:8, :] = jnp.zeros((6, D), jnp.float32)


def _scale_shift(stats, gamma, beta):
    tot = jnp.sum(stats, axis=0)
    mean = tot[0:1, :] * (1.0 / N)
    ex2 = tot[1:2, :] * (1.0 / N)
    var = ex2 - mean * mean
    scale = gamma / jnp.sqrt(var + EPS)
    shift = beta - mean * scale
    return scale, shift


BE = B * NN


NG = B // 8


def _conv_weighted(gpos_ref, qrep_ref, m_ref, kp2_ref, sel_ref, w_ref,
                   feats, out_ref, stats_ref, ws_ref):
    qrep = jnp.broadcast_to(qrep_ref[...][:, None, :],
                            (B, NN, 16)).reshape(BE, 16)
    rel = gpos_ref[...] - qrep
    lhs = jnp.concatenate([rel * rel, rel], axis=1)
    d2 = jnp.dot(lhs, m_ref[...],
                 preferred_element_type=jnp.float32) + kp2_ref[...]
    d2c = jnp.maximum(d2, 1e-24)
    dist = d2c * lax.rsqrt(d2c)
    infl = jnp.maximum(0.0, 1.0 - dist)
    mask = sel_ref[...]
    for g in range(NG):
        st = infl[g * 128:(g + 1) * 128, :] * mask
        fg = feats[g * 128:(g + 1) * 128, :]
        wg = jax.lax.dot_general(
            st, fg, (((0,), (0,)), ((), ())),
            preferred_element_type=jnp.float32)
        for k in range(K):
            ws_ref[k, g * 8:(g + 1) * 8, :] = wg[k * 8:(k + 1) * 8, :]
    acc = jnp.dot(ws_ref[0], w_ref[0], preferred_element_type=jnp.float32)
    for k in range(1, K):
        acc = acc + jnp.dot(ws_ref[k], w_ref[k],
                            preferred_element_type=jnp.float32)
    out_ref[...] = acc
    _write_stats(stats_ref, acc)


def _conv1_body(gpos_ref, qrep_ref, gfeat_ref, m_ref, kp2_ref, sel_ref, w_ref,
                out_ref, stats_ref, ws_ref):
    _conv_weighted(gpos_ref, qrep_ref, m_ref, kp2_ref, sel_ref, w_ref,
                   gfeat_ref[...], out_ref, stats_ref, ws_ref)


def _conv2_body(gpos_ref, qrep_ref, gfeat_ref, m_ref, kp2_ref, sel_ref, w_ref,
                stats_in_ref, gamma_ref, beta_ref,
                out_ref, stats_ref, ws_ref):
    scale, shift = _scale_shift(stats_in_ref[...], gamma_ref[...], beta_ref[...])
    feats = _leaky(gfeat_ref[...] * scale + shift)
    _conv_weighted(gpos_ref, qrep_ref, m_ref, kp2_ref, sel_ref, w_ref,
                   feats, out_ref, stats_ref, ws_ref)


_CONV_COMMON_SPECS = [
    pl.BlockSpec((BE, 16), lambda i: (i, 0)),
    pl.BlockSpec((B, 16), lambda i: (i, 0)),
    pl.BlockSpec((BE, D), lambda i: (i, 0)),
    pl.BlockSpec((32, D), lambda i: (0, 0)),
    pl.BlockSpec((1, D), lambda i: (0, 0)),
    pl.BlockSpec((D, D), lambda i: (0, 0)),
    pl.BlockSpec((K, D, D), lambda i: (0, 0, 0)),
]

_CONV_OUT_SPECS = [
    pl.BlockSpec((B, D), lambda i: (i, 0)),
    pl.BlockSpec((1, 8, D), lambda i: (i, 0, 0)),
]

_CONV_OUT_SHAPE = [
    jax.ShapeDtypeStruct((N, D), jnp.float32),
    jax.ShapeDtypeStruct((NBLK, 8, D), jnp.float32),
]


def _tc_conv1(gpos, qrep, gfeat, m, kp2, sel, wflat):
    return pl.pallas_call(
        _conv1_body,
        grid=(NBLK,),
        in_specs=_CONV_COMMON_SPECS,
        out_specs=_CONV_OUT_SPECS,
        out_shape=_CONV_OUT_SHAPE,
        scratch_shapes=[pltpu.VMEM((K, B, D), jnp.float32)],
    )(gpos, qrep, gfeat, m, kp2, sel, wflat)


def _tc_conv2(gpos, qrep, gfeat, m, kp2, sel, wflat, stats_in, gamma, beta):
    return pl.pallas_call(
        _conv2_body,
        grid=(NBLK,),
        in_specs=_CONV_COMMON_SPECS + [
            pl.BlockSpec((NBLK, 8, D), lambda i: (0, 0, 0)),
            pl.BlockSpec((1, D), lambda i: (0, 0)),
            pl.BlockSpec((1, D), lambda i: (0, 0)),
        ],
        out_specs=_CONV_OUT_SPECS,
        out_shape=_CONV_OUT_SHAPE,
        scratch_shapes=[pltpu.VMEM((K, B, D), jnp.float32)],
    )(gpos, qrep, gfeat, m, kp2, sel, wflat, stats_in, gamma, beta)


def _final_body(h_ref, x_ref, stats_in_ref, gamma_ref, beta_ref, out_ref):
    scale, shift = _scale_shift(stats_in_ref[...], gamma_ref[...], beta_ref[...])
    out_ref[...] = _leaky(h_ref[...] * scale + shift) + x_ref[...]


def _tc_final(h, x, stats_in, gamma, beta):
    return pl.pallas_call(
        _final_body,
        grid=(N // B3,),
        in_specs=[
            pl.BlockSpec((B3, D), lambda i: (i, 0)),
            pl.BlockSpec((B3, D), lambda i: (i, 0)),
            pl.BlockSpec((NBLK, 8, D), lambda i: (0, 0, 0)),
            pl.BlockSpec((1, D), lambda i: (0, 0)),
            pl.BlockSpec((1, D), lambda i: (0, 0)),
        ],
        out_specs=pl.BlockSpec((B3, D), lambda i: (i, 0)),
        out_shape=jax.ShapeDtypeStruct((N, D), jnp.float32),
    )(h, x, stats_in, gamma, beta)


def kernel(x, points, neighbors, kernel_points, W0, W1,
           gamma0, beta0, gamma1, beta1):
    idx = neighbors.astype(jnp.int32).reshape(-1)
    qpts = jnp.pad(points, ((0, 0), (0, 13)))
    kp = jnp.pad(kernel_points, ((0, 1), (0, 13)))
    m = jnp.concatenate([jnp.ones((16, 16), jnp.float32), -2.0 * kp.T], axis=0)
    m = jnp.repeat(m, 8, axis=1)
    kp2 = jnp.repeat(jnp.sum(kp * kp, axis=1).reshape(1, 16), 8, axis=1)
    lane = jnp.arange(D)
    edge = jnp.arange(D)
    sel = ((lane[None, :] % 8 == edge[:, None] // NN) &
           (lane[None, :] // 8 < K)).astype(jnp.float32)
    w0f = W0
    w1f = W1
    g0 = gamma0.reshape(1, D)
    b0 = beta0.reshape(1, D)
    g1 = gamma1.reshape(1, D)
    b1 = beta1.reshape(1, D)

    gather_l1, gather_l2 = _sc_gathers()
    gfeat0, gpos = gather_l1(x, qpts, idx)

    h1_raw, stats1 = _tc_conv1(gpos, qpts, gfeat0, m, kp2, sel, w0f)

    gfeat1 = gather_l2(h1_raw, idx)

    h2_raw, stats2 = _tc_conv2(gpos, qpts, gfeat1, m, kp2, sel, w1f,
                               stats1, g0, b0)

    return _tc_final(h2_raw, x, stats2, g1, b1)


# speedup vs baseline: 1.0602x; 1.0292x over previous
import functools

import jax
import jax.numpy as jnp
from jax import lax
from jax.experimental import pallas as pl
from jax.experimental.pallas import tpu as pltpu
from jax.experimental.pallas import tpu_sc as plsc

N = 10000
NN = 16
D = 128
K = 15
RADIUS = 1.0
EPS = 1e-5
NEG_SLOPE = 0.2

NE = N * NN
B = 400
NBLK = N // B
B3 = 1000
CHUNK = 200


def _leaky(v):
    return jnp.where(v >= 0, v, NEG_SLOPE * v)


@functools.cache
def _sc_gathers():
    info = plsc.get_sparse_core_info()
    nc = info.num_cores
    nw = nc * info.num_subcores
    bpw = NE // nw
    nchunk = bpw // CHUNK
    mesh = plsc.VectorSubcoreMesh(core_axis_name="c", subcore_axis_name="s")

    def pipeline(idx_hbm, idx_v, sem_i, tables, wid):
        pltpu.async_copy(idx_hbm.at[pl.ds(wid * bpw, bpw)], idx_v,
                         sem_i).wait()

        def gather(t):
            r = t % 3
            hs = []
            for (src, _out, bufs, gs, _wsems) in tables:
                hs.append(pltpu.async_copy(
                    src.at[idx_v.at[pl.ds(t * CHUNK, CHUNK)]], bufs[r],
                    gs[r]))
            return hs

        g = {0: gather(0), 1: gather(1)}
        w = {}
        for t in range(nchunk):
            r = t % 3
            if t + 2 < nchunk:
                if t >= 1:
                    for h in w[t - 1]:
                        h.wait()
                g[t + 2] = gather(t + 2)
            for h in g[t]:
                h.wait()
            base = wid * bpw + t * CHUNK
            w[t] = [
                pltpu.async_copy(bufs[r], out.at[pl.ds(base, CHUNK)],
                                 wsems[r])
                for (_src, out, bufs, _gs, wsems) in tables
            ]
        for t in (nchunk - 2, nchunk - 1):
            for h in w[t]:
                h.wait()

    @functools.partial(
        pl.kernel,
        mesh=mesh,
        compiler_params=pltpu.CompilerParams(use_tc_tiling_on_sc=False),
        out_type=jax.ShapeDtypeStruct((NE, 16), jnp.float32),
        scratch_types=(
            [pltpu.VMEM((bpw,), jnp.int32)]
            + [pltpu.VMEM((CHUNK, 16), jnp.float32)] * 3
            + [pltpu.SemaphoreType.DMA] * 7
        ),
    )
    def gather_pos(pos_hbm, idx_hbm, pos_out,
                   idx_v, p0, p1, p2,
                   sem_i, gp0, gp1, gp2, wp0, wp1, wp2):
        wid = lax.axis_index("s") * nc + lax.axis_index("c")
        pipeline(idx_hbm, idx_v, sem_i,
                 [(pos_hbm, pos_out, [p0, p1, p2],
                   [gp0, gp1, gp2], [wp0, wp1, wp2])],
                 wid)

    @functools.partial(
        pl.kernel,
        mesh=mesh,
        out_type=jax.ShapeDtypeStruct((NE, D), jnp.float32),
        scratch_types=(
            [pltpu.VMEM((bpw,), jnp.int32)]
            + [pltpu.VMEM((CHUNK, D), jnp.float32)] * 3
            + [pltpu.SemaphoreType.DMA] * 7
        ),
    )
    def gather_l2(feat_hbm, idx_hbm, feat_out,
                  idx_v, f0, f1, f2,
                  sem_i, gf0, gf1, gf2, wf0, wf1, wf2):
        wid = lax.axis_index("s") * nc + lax.axis_index("c")
        pipeline(idx_hbm, idx_v, sem_i,
                 [(feat_hbm, feat_out, [f0, f1, f2],
                   [gf0, gf1, gf2], [wf0, wf1, wf2])],
                 wid)

    return gather_pos, gather_l2


def _write_stats(stats_ref, acc):
    stats_ref[0, 0:1, :] = jnp.sum(acc, axis=0, keepdims=True)
    stats_ref[0, 1:2, :] = jnp.sum(acc * acc, axis=0, keepdims=True)
    stats_ref[0, 2:8, :] = jnp.zeros((6, D), jnp.float32)


def _scale_shift(stats, gamma, beta):
    tot = jnp.sum(stats, axis=0)
    mean = tot[0:1, :] * (1.0 / N)
    ex2 = tot[1:2, :] * (1.0 / N)
    var = ex2 - mean * mean
    scale = gamma / jnp.sqrt(var + EPS)
    shift = beta - mean * scale
    return scale, shift


BE = B * NN


NG = B // 8


def _conv_weighted(gpos_ref, qrep_ref, m_ref, kp2_ref, sel_ref, w_ref,
                   feats, out_ref, stats_ref, ws_ref):
    qrep = jnp.broadcast_to(qrep_ref[...][:, None, :],
                            (B, NN, 16)).reshape(BE, 16)
    rel = gpos_ref[...] - qrep
    lhs = jnp.concatenate([rel * rel, rel], axis=1)
    d2 = jnp.dot(lhs, m_ref[...],
                 preferred_element_type=jnp.float32) + kp2_ref[...]
    d2c = jnp.maximum(d2, 1e-24)
    dist = d2c * lax.rsqrt(d2c)
    infl = jnp.maximum(0.0, 1.0 - dist)
    mask = sel_ref[...]
    for g in range(NG):
        st = infl[g * 128:(g + 1) * 128, :] * mask
        fg = feats[g * 128:(g + 1) * 128, :]
        wg = jax.lax.dot_general(
            st, fg, (((0,), (0,)), ((), ())),
            preferred_element_type=jnp.float32)
        for k in range(K):
            ws_ref[k, g * 8:(g + 1) * 8, :] = wg[k * 8:(k + 1) * 8, :]
    acc = jnp.dot(ws_ref[0], w_ref[0], preferred_element_type=jnp.float32)
    for k in range(1, K):
        acc = acc + jnp.dot(ws_ref[k], w_ref[k],
                            preferred_element_type=jnp.float32)
    out_ref[...] = acc
    _write_stats(stats_ref, acc)


def _conv1_body(gpos_ref, qrep_ref, gfeat_ref, m_ref, kp2_ref, sel_ref, w_ref,
                out_ref, stats_ref, ws_ref):
    _conv_weighted(gpos_ref, qrep_ref, m_ref, kp2_ref, sel_ref, w_ref,
                   gfeat_ref[...], out_ref, stats_ref, ws_ref)


def _conv2_body(gpos_ref, qrep_ref, gfeat_ref, m_ref, kp2_ref, sel_ref, w_ref,
                stats_in_ref, gamma_ref, beta_ref,
                out_ref, stats_ref, ws_ref):
    scale, shift = _scale_shift(stats_in_ref[...], gamma_ref[...], beta_ref[...])
    feats = _leaky(gfeat_ref[...] * scale + shift)
    _conv_weighted(gpos_ref, qrep_ref, m_ref, kp2_ref, sel_ref, w_ref,
                   feats, out_ref, stats_ref, ws_ref)


_CONV_COMMON_SPECS = [
    pl.BlockSpec((BE, 16), lambda i: (i, 0)),
    pl.BlockSpec((B, 16), lambda i: (i, 0)),
    pl.BlockSpec((BE, D), lambda i: (i, 0)),
    pl.BlockSpec((32, D), lambda i: (0, 0)),
    pl.BlockSpec((1, D), lambda i: (0, 0)),
    pl.BlockSpec((D, D), lambda i: (0, 0)),
    pl.BlockSpec((K, D, D), lambda i: (0, 0, 0)),
]

_CONV_OUT_SPECS = [
    pl.BlockSpec((B, D), lambda i: (i, 0)),
    pl.BlockSpec((1, 8, D), lambda i: (i, 0, 0)),
]

_CONV_OUT_SHAPE = [
    jax.ShapeDtypeStruct((N, D), jnp.float32),
    jax.ShapeDtypeStruct((NBLK, 8, D), jnp.float32),
]


def _tc_conv1(gpos, qrep, gfeat, m, kp2, sel, wflat):
    return pl.pallas_call(
        _conv1_body,
        grid=(NBLK,),
        in_specs=_CONV_COMMON_SPECS,
        out_specs=_CONV_OUT_SPECS,
        out_shape=_CONV_OUT_SHAPE,
        scratch_shapes=[pltpu.VMEM((K, B, D), jnp.float32)],
    )(gpos, qrep, gfeat, m, kp2, sel, wflat)


def _tc_conv2(gpos, qrep, gfeat, m, kp2, sel, wflat, stats_in, gamma, beta):
    return pl.pallas_call(
        _conv2_body,
        grid=(NBLK,),
        in_specs=_CONV_COMMON_SPECS + [
            pl.BlockSpec((NBLK, 8, D), lambda i: (0, 0, 0)),
            pl.BlockSpec((1, D), lambda i: (0, 0)),
            pl.BlockSpec((1, D), lambda i: (0, 0)),
        ],
        out_specs=_CONV_OUT_SPECS,
        out_shape=_CONV_OUT_SHAPE,
        scratch_shapes=[pltpu.VMEM((K, B, D), jnp.float32)],
    )(gpos, qrep, gfeat, m, kp2, sel, wflat, stats_in, gamma, beta)


def _final_body(h_ref, x_ref, stats_in_ref, gamma_ref, beta_ref, out_ref):
    scale, shift = _scale_shift(stats_in_ref[...], gamma_ref[...], beta_ref[...])
    out_ref[...] = _leaky(h_ref[...] * scale + shift) + x_ref[...]


def _tc_final(h, x, stats_in, gamma, beta):
    return pl.pallas_call(
        _final_body,
        grid=(N // B3,),
        in_specs=[
            pl.BlockSpec((B3, D), lambda i: (i, 0)),
            pl.BlockSpec((B3, D), lambda i: (i, 0)),
            pl.BlockSpec((NBLK, 8, D), lambda i: (0, 0, 0)),
            pl.BlockSpec((1, D), lambda i: (0, 0)),
            pl.BlockSpec((1, D), lambda i: (0, 0)),
        ],
        out_specs=pl.BlockSpec((B3, D), lambda i: (i, 0)),
        out_shape=jax.ShapeDtypeStruct((N, D), jnp.float32),
    )(h, x, stats_in, gamma, beta)


def kernel(x, points, neighbors, kernel_points, W0, W1,
           gamma0, beta0, gamma1, beta1):
    idx = neighbors.astype(jnp.int32).reshape(-1)
    qpts = jnp.pad(points, ((0, 0), (0, 13)))
    kp = jnp.pad(kernel_points, ((0, 1), (0, 13)))
    m = jnp.concatenate([jnp.ones((16, 16), jnp.float32), -2.0 * kp.T], axis=0)
    m = jnp.repeat(m, 8, axis=1)
    kp2 = jnp.repeat(jnp.sum(kp * kp, axis=1).reshape(1, 16), 8, axis=1)
    lane = jnp.arange(D)
    edge = jnp.arange(D)
    sel = ((lane[None, :] % 8 == edge[:, None] // NN) &
           (lane[None, :] // 8 < K)).astype(jnp.float32)
    w0f = W0
    w1f = W1
    g0 = gamma0.reshape(1, D)
    b0 = beta0.reshape(1, D)
    g1 = gamma1.reshape(1, D)
    b1 = beta1.reshape(1, D)

    gather_pos, gather_feat = _sc_gathers()
    gpos = gather_pos(qpts, idx)
    gfeat0 = gather_feat(x, idx)

    h1_raw, stats1 = _tc_conv1(gpos, qpts, gfeat0, m, kp2, sel, w0f)

    gfeat1 = gather_feat(h1_raw, idx)

    h2_raw, stats2 = _tc_conv2(gpos, qpts, gfeat1, m, kp2, sel, w1f,
                               stats1, g0, b0)

    return _tc_final(h2_raw, x, stats2, g1, b1)
